# Initial kernel scaffold; baseline (speedup 1.0000x reference)
import jax
import jax.numpy as jnp
from jax.experimental import pallas as pl


def kernel(node_features, edge_index, W_node, b_node, att_w, W_out, b_out):
    raise NotImplementedError("write your pallas kernel here")


# speedup vs baseline: 4.9023x; 4.9023x over previous
import jax
import jax.numpy as jnp
from jax import lax
from jax.experimental import pallas as pl
from jax.experimental.pallas import tpu as pltpu

N = 10000
E = 320000
HID = 128
H = 8
HD = HID // H

NT = 10240
NC = 2048
EB = 512
NEB = E // EB


def _body(nf_ref, wa_ref, ba_ref, r_ref, c_ref, msk_ref, e8_ref,
          wo_ref, bo_ref, out_ref, a_ref, m_ref, acc_ref, den_ref):
    i = pl.program_id(0)
    f32 = jnp.float32

    @pl.when(i == 0)
    def _():
        a = jnp.dot(nf_ref[...], wa_ref[...], preferred_element_type=f32)
        a = a + ba_ref[0:1, :]
        a_ref[...] = a
        cm = jnp.max(a, axis=0, keepdims=True)
        sw = jnp.concatenate([cm[:, H:], cm[:, :H]], axis=1)
        m_ref[...] = jnp.maximum(cm + sw, 0.0)
        acc_ref[...] = jnp.zeros((NT, HID), f32)
        den_ref[...] = jnp.zeros((NT, 2 * H), f32)

    r2 = r_ref[...]
    c2 = c_ref[...]

    ag = jnp.zeros((EB, 2 * H), f32)
    cg = jnp.zeros((EB, 2 * H), f32)
    xg = jnp.zeros((EB, HID), f32)
    for nc in range(NT // NC):
        ids = lax.broadcasted_iota(jnp.int32, (EB, NC), 1) + nc * NC
        ohr = (r2 == ids).astype(f32)
        ohc = (c2 == ids).astype(f32)
        sl = pl.ds(nc * NC, NC)
        ag = ag + jnp.dot(ohr, a_ref[sl, :], preferred_element_type=f32)
        cg = cg + jnp.dot(ohc, a_ref[sl, :], preferred_element_type=f32)
        xg = xg + jnp.dot(ohr, nf_ref[sl, :], preferred_element_type=f32)

    csw = jnp.concatenate([cg[:, H:], cg[:, :H]], axis=1)
    s = (ag + csw) * msk_ref[0:1, :]
    s = jnp.maximum(s, 0.2 * s)
    p = jnp.exp(s - m_ref[0:1, :])
    pe = jnp.dot(p, e8_ref[...], preferred_element_type=f32)
    w = pe * xg

    for nc in range(NT // NC):
        ids = lax.broadcasted_iota(jnp.int32, (EB, NC), 1) + nc * NC
        ohc = (c2 == ids).astype(f32)
        sl = pl.ds(nc * NC, NC)
        dn = ((0,), (0,)), ((), ())
        den_ref[sl, :] = den_ref[sl, :] + lax.dot_general(
            ohc, p, dimension_numbers=dn, preferred_element_type=f32)
        acc_ref[sl, :] = acc_ref[sl, :] + lax.dot_general(
            ohc, w, dimension_numbers=dn, preferred_element_type=f32)

    @pl.when(i == pl.num_programs(0) - 1)
    def _():
        d = jnp.dot(den_ref[...], e8_ref[...], preferred_element_type=f32)
        y = acc_ref[...] / (d + 1e-10)
        out_ref[...] = (
            jnp.dot(y, wo_ref[...], preferred_element_type=f32)
            + bo_ref[0:1, :]
        )


def kernel(node_features, edge_index, W_node, b_node, att_w, W_out, b_out):
    f32 = jnp.float32
    nf = node_features.astype(f32)

    wn = W_node.reshape(HID, H, 2 * HD)
    w_s = jnp.einsum("khj,hj->kh", wn, att_w[:, : 2 * HD])
    w_d = jnp.einsum("khj,hj->kh", wn, att_w[:, 2 * HD:])
    w_a = jnp.concatenate([w_s, w_d], axis=1)
    bn = b_node.reshape(H, 2 * HD)
    b_s = jnp.sum(bn * att_w[:, : 2 * HD], axis=-1)
    b_d = jnp.sum(bn * att_w[:, 2 * HD:], axis=-1)
    b_a8 = jnp.broadcast_to(jnp.concatenate([b_s, b_d])[None, :], (8, 2 * H))

    nf_pad = jnp.pad(nf, ((0, NT - N), (0, 0)))
    row = edge_index[0].astype(jnp.int32).reshape(E, 1)
    col = edge_index[1].astype(jnp.int32).reshape(E, 1)

    mask = jnp.broadcast_to(
        (jnp.arange(2 * H) < H).astype(f32)[None, :], (8, 2 * H))
    e8 = jnp.concatenate(
        [jnp.repeat(jnp.eye(H, dtype=f32), HD, axis=1),
         jnp.zeros((H, HID), f32)], axis=0)
    b_out8 = jnp.broadcast_to(b_out.astype(f32)[None, :], (8, HID))

    out = pl.pallas_call(
        _body,
        grid=(NEB,),
        in_specs=[
            pl.BlockSpec((NT, HID), lambda i: (0, 0)),
            pl.BlockSpec((HID, 2 * H), lambda i: (0, 0)),
            pl.BlockSpec((8, 2 * H), lambda i: (0, 0)),
            pl.BlockSpec((EB, 1), lambda i: (i, 0)),
            pl.BlockSpec((EB, 1), lambda i: (i, 0)),
            pl.BlockSpec((8, 2 * H), lambda i: (0, 0)),
            pl.BlockSpec((2 * H, HID), lambda i: (0, 0)),
            pl.BlockSpec((HID, HID), lambda i: (0, 0)),
            pl.BlockSpec((8, HID), lambda i: (0, 0)),
        ],
        out_specs=pl.BlockSpec((NT, HID), lambda i: (0, 0)),
        out_shape=jax.ShapeDtypeStruct((NT, HID), f32),
        scratch_shapes=[
            pltpu.VMEM((NT, 2 * H), f32),
            pltpu.VMEM((1, 2 * H), f32),
            pltpu.VMEM((NT, HID), f32),
            pltpu.VMEM((NT, 2 * H), f32),
        ],
    )(nf_pad, w_a, b_a8, row, col, mask, e8, W_out.astype(f32), b_out8)
    return out[:N]
